# baseline (device time: 11776 ns/iter reference)
import jax
import jax.numpy as jnp
from jax import lax
from jax.experimental import pallas as pl
from jax.experimental.pallas import tpu as pltpu

N_DEV = 4


def kernel(x, w_mat):
    m_per, k = x.shape
    _, n = w_mat.shape
    n_per = n // N_DEV

    def body(x_hbm, w_hbm, out_ref, x_v, w_v, send_v, recv_v,
             load_sems, send_sems, recv_sems):
        my = lax.axis_index("i")

        def w_load(idx):
            j = (my + idx + 1) % N_DEV
            cp = pltpu.make_async_copy(
                w_hbm.at[:, pl.ds(j * n_per, n_per)],
                w_v.at[idx],
                load_sems.at[1 + idx],
            )
            cp.start()
            return cp

        cp_x = pltpu.make_async_copy(x_hbm, x_v, load_sems.at[0])
        cp_x.start()
        w_cps = [w_load(0)]

        barrier_sem = pltpu.get_barrier_semaphore()
        for o in range(1, N_DEV):
            pl.semaphore_signal(
                barrier_sem, inc=1,
                device_id=((my + o) % N_DEV,),
                device_id_type=pl.DeviceIdType.MESH,
            )

        cp_x.wait()
        rdmas = []
        for idx in range(N_DEV - 1):
            tgt = (my + idx + 1) % N_DEV
            w_cps[idx].wait()
            w_cps.append(w_load(idx + 1))
            yb = jnp.dot(x_v[:, :], w_v[idx, :, :],
                         preferred_element_type=jnp.float32)
            send_v[idx, :, :] = (yb * jax.nn.sigmoid(yb)).astype(jnp.bfloat16)
            if idx == 0:
                pl.semaphore_wait(barrier_sem, N_DEV - 1)
            rdma = pltpu.make_async_remote_copy(
                src_ref=send_v.at[idx],
                dst_ref=recv_v.at[idx],
                send_sem=send_sems.at[idx],
                recv_sem=recv_sems.at[idx],
                device_id=(tgt,),
                device_id_type=pl.DeviceIdType.MESH,
            )
            rdma.start()
            rdmas.append(rdma)

        w_cps[N_DEV - 1].wait()
        yb = jnp.dot(x_v[:, :], w_v[N_DEV - 1, :, :],
                     preferred_element_type=jnp.float32)
        out_ref[pl.ds(my * m_per, m_per), :] = yb * jax.nn.sigmoid(yb)

        for idx in range(N_DEV - 1):
            src = (my - idx - 1) % N_DEV
            rdmas[idx].wait_recv()
            out_ref[pl.ds(src * m_per, m_per), :] = recv_v[idx].astype(
                jnp.float32)
        for idx in range(N_DEV - 1):
            rdmas[idx].wait_send()

    x = pltpu.with_memory_space_constraint(x, pltpu.MemorySpace.HBM)
    w_mat = pltpu.with_memory_space_constraint(w_mat, pltpu.MemorySpace.HBM)
    return pl.pallas_call(
        body,
        out_shape=jax.ShapeDtypeStruct((N_DEV * m_per, n_per), jnp.float32),
        in_specs=[
            pl.BlockSpec(memory_space=pltpu.MemorySpace.HBM),
            pl.BlockSpec(memory_space=pltpu.MemorySpace.HBM),
        ],
        out_specs=pl.BlockSpec(memory_space=pltpu.VMEM),
        scratch_shapes=[
            pltpu.VMEM((m_per, k), jnp.float32),
            pltpu.VMEM((N_DEV, k, n_per), jnp.float32),
            pltpu.VMEM((N_DEV - 1, m_per, n_per), jnp.bfloat16),
            pltpu.VMEM((N_DEV - 1, m_per, n_per), jnp.bfloat16),
            pltpu.SemaphoreType.DMA((1 + N_DEV,)),
            pltpu.SemaphoreType.DMA((N_DEV - 1,)),
            pltpu.SemaphoreType.DMA((N_DEV - 1,)),
        ],
        compiler_params=pltpu.CompilerParams(collective_id=0),
    )(x, w_mat)


# device time: 10672 ns/iter; 1.1034x vs baseline; 1.1034x over previous
import jax
import jax.numpy as jnp
from jax import lax
from jax.experimental import pallas as pl
from jax.experimental.pallas import tpu as pltpu

N_DEV = 4


def kernel(x, w_mat):
    m_per, k = x.shape
    _, n = w_mat.shape
    n_per = n // N_DEV

    def body(x_hbm, w_hbm, out_ref, x_v, w_v, send_v, recv_v,
             load_sems, send_sems, recv_sems):
        my = lax.axis_index("i")

        def w_load(idx):
            j = (my + idx + 1) % N_DEV
            cp = pltpu.make_async_copy(
                w_hbm.at[:, pl.ds(j * n_per, n_per)],
                w_v.at[idx],
                load_sems.at[1 + idx],
            )
            cp.start()
            return cp

        cp_x = pltpu.make_async_copy(x_hbm, x_v, load_sems.at[0])
        cp_x.start()
        w_cps = [w_load(idx) for idx in range(N_DEV)]

        barrier_sem = pltpu.get_barrier_semaphore()
        for o in range(1, N_DEV):
            pl.semaphore_signal(
                barrier_sem, inc=1,
                device_id=((my + o) % N_DEV,),
                device_id_type=pl.DeviceIdType.MESH,
            )

        cp_x.wait()
        rdmas = []
        for idx in range(N_DEV - 1):
            tgt = (my + idx + 1) % N_DEV
            w_cps[idx].wait()
            yb = jnp.dot(x_v[:, :], w_v[idx, :, :],
                         preferred_element_type=jnp.float32)
            send_v[idx, :, :] = (yb * jax.nn.sigmoid(yb)).astype(jnp.bfloat16)
            if idx == 0:
                pl.semaphore_wait(barrier_sem, N_DEV - 1)
            rdma = pltpu.make_async_remote_copy(
                src_ref=send_v.at[idx],
                dst_ref=recv_v.at[idx],
                send_sem=send_sems.at[idx],
                recv_sem=recv_sems.at[idx],
                device_id=(tgt,),
                device_id_type=pl.DeviceIdType.MESH,
            )
            rdma.start()
            rdmas.append(rdma)

        w_cps[N_DEV - 1].wait()
        yb = jnp.dot(x_v[:, :], w_v[N_DEV - 1, :, :],
                     preferred_element_type=jnp.float32)
        out_ref[pl.ds(my * m_per, m_per), :] = yb * jax.nn.sigmoid(yb)

        for idx in range(N_DEV - 1):
            src = (my - idx - 1) % N_DEV
            rdmas[idx].wait_recv()
            out_ref[pl.ds(src * m_per, m_per), :] = recv_v[idx].astype(
                jnp.float32)
        for idx in range(N_DEV - 1):
            rdmas[idx].wait_send()

    x = pltpu.with_memory_space_constraint(x, pltpu.MemorySpace.HBM)
    w_mat = pltpu.with_memory_space_constraint(w_mat, pltpu.MemorySpace.HBM)
    return pl.pallas_call(
        body,
        out_shape=jax.ShapeDtypeStruct((N_DEV * m_per, n_per), jnp.float32),
        in_specs=[
            pl.BlockSpec(memory_space=pltpu.MemorySpace.HBM),
            pl.BlockSpec(memory_space=pltpu.MemorySpace.HBM),
        ],
        out_specs=pl.BlockSpec(memory_space=pltpu.VMEM),
        scratch_shapes=[
            pltpu.VMEM((m_per, k), jnp.float32),
            pltpu.VMEM((N_DEV, k, n_per), jnp.float32),
            pltpu.VMEM((N_DEV - 1, m_per, n_per), jnp.bfloat16),
            pltpu.VMEM((N_DEV - 1, m_per, n_per), jnp.bfloat16),
            pltpu.SemaphoreType.DMA((1 + N_DEV,)),
            pltpu.SemaphoreType.DMA((N_DEV - 1,)),
            pltpu.SemaphoreType.DMA((N_DEV - 1,)),
        ],
        compiler_params=pltpu.CompilerParams(collective_id=0),
    )(x, w_mat)


# device time: 10659 ns/iter; 1.1048x vs baseline; 1.0012x over previous
import jax
import jax.numpy as jnp
from jax import lax
from jax.experimental import pallas as pl
from jax.experimental.pallas import tpu as pltpu

N_DEV = 4


def kernel(x, w_mat):
    m_per, k = x.shape
    _, n = w_mat.shape
    n_per = n // N_DEV

    def body(x_hbm, w_hbm, out_ref, x_v, w_v, send_v, recv_v,
             load_sems, send_sems, recv_sems):
        my = lax.axis_index("i")

        kh = k // 2
        j0 = (my + 1) % N_DEV
        cp_xa = pltpu.make_async_copy(
            x_hbm.at[:, pl.ds(0, kh)], x_v.at[:, pl.ds(0, kh)],
            load_sems.at[0])
        cp_xb = pltpu.make_async_copy(
            x_hbm.at[:, pl.ds(kh, kh)], x_v.at[:, pl.ds(kh, kh)],
            load_sems.at[1])
        cp_w0a = pltpu.make_async_copy(
            w_hbm.at[pl.ds(0, kh), pl.ds(j0 * n_per, n_per)],
            w_v.at[0, pl.ds(0, kh), :], load_sems.at[2])
        cp_w0b = pltpu.make_async_copy(
            w_hbm.at[pl.ds(kh, kh), pl.ds(j0 * n_per, n_per)],
            w_v.at[0, pl.ds(kh, kh), :], load_sems.at[3])
        for cp in (cp_xa, cp_w0a, cp_xb, cp_w0b):
            cp.start()
        w_cps = [None]
        for idx in range(1, N_DEV):
            j = (my + idx + 1) % N_DEV
            cp = pltpu.make_async_copy(
                w_hbm.at[:, pl.ds(j * n_per, n_per)],
                w_v.at[idx],
                load_sems.at[3 + idx],
            )
            cp.start()
            w_cps.append(cp)

        barrier_sem = pltpu.get_barrier_semaphore()
        for o in range(1, N_DEV):
            pl.semaphore_signal(
                barrier_sem, inc=1,
                device_id=((my + o) % N_DEV,),
                device_id_type=pl.DeviceIdType.MESH,
            )

        rdmas = []
        for idx in range(N_DEV - 1):
            tgt = (my + idx + 1) % N_DEV
            if idx == 0:
                cp_xa.wait()
                cp_w0a.wait()
                part = jnp.dot(x_v[:, :kh], w_v[0, :kh, :],
                               preferred_element_type=jnp.float32)
                cp_xb.wait()
                cp_w0b.wait()
                yb = part + jnp.dot(x_v[:, kh:], w_v[0, kh:, :],
                                    preferred_element_type=jnp.float32)
            else:
                w_cps[idx].wait()
                yb = jnp.dot(x_v[:, :], w_v[idx, :, :],
                             preferred_element_type=jnp.float32)
            send_v[idx, :, :] = (yb * jax.nn.sigmoid(yb)).astype(jnp.bfloat16)
            if idx == 0:
                pl.semaphore_wait(barrier_sem, N_DEV - 1)
            rdma = pltpu.make_async_remote_copy(
                src_ref=send_v.at[idx],
                dst_ref=recv_v.at[idx],
                send_sem=send_sems.at[idx],
                recv_sem=recv_sems.at[idx],
                device_id=(tgt,),
                device_id_type=pl.DeviceIdType.MESH,
            )
            rdma.start()
            rdmas.append(rdma)

        w_cps[N_DEV - 1].wait()
        yb = jnp.dot(x_v[:, :], w_v[N_DEV - 1, :, :],
                     preferred_element_type=jnp.float32)
        out_ref[pl.ds(my * m_per, m_per), :] = yb * jax.nn.sigmoid(yb)

        for idx in range(N_DEV - 1):
            src = (my - idx - 1) % N_DEV
            rdmas[idx].wait_recv()
            out_ref[pl.ds(src * m_per, m_per), :] = recv_v[idx].astype(
                jnp.float32)
        for idx in range(N_DEV - 1):
            rdmas[idx].wait_send()

    x = pltpu.with_memory_space_constraint(x, pltpu.MemorySpace.HBM)
    w_mat = pltpu.with_memory_space_constraint(w_mat, pltpu.MemorySpace.HBM)
    return pl.pallas_call(
        body,
        out_shape=jax.ShapeDtypeStruct((N_DEV * m_per, n_per), jnp.float32),
        in_specs=[
            pl.BlockSpec(memory_space=pltpu.MemorySpace.HBM),
            pl.BlockSpec(memory_space=pltpu.MemorySpace.HBM),
        ],
        out_specs=pl.BlockSpec(memory_space=pltpu.VMEM),
        scratch_shapes=[
            pltpu.VMEM((m_per, k), jnp.float32),
            pltpu.VMEM((N_DEV, k, n_per), jnp.float32),
            pltpu.VMEM((N_DEV - 1, m_per, n_per), jnp.bfloat16),
            pltpu.VMEM((N_DEV - 1, m_per, n_per), jnp.bfloat16),
            pltpu.SemaphoreType.DMA((3 + N_DEV,)),
            pltpu.SemaphoreType.DMA((N_DEV - 1,)),
            pltpu.SemaphoreType.DMA((N_DEV - 1,)),
        ],
        compiler_params=pltpu.CompilerParams(collective_id=0),
    )(x, w_mat)
